# baseline (speedup 1.0000x reference)
import functools

import jax
import jax.numpy as jnp
from jax import lax
from jax.experimental import pallas as pl
from jax.experimental.pallas import tpu as pltpu
from jax.experimental.pallas import tpu_sc as plsc

NUM_TEAMS = 1000000
EMBED_DIM = 16
NUM_FEATURES = 22
BATCH = 16384

NC, NS = 2, 16
NW = NC * NS
CHUNK = 128
B2 = 2 * BATCH
D_SC = EMBED_DIM // NC
T_W = B2 // NS
KJ = T_W // CHUNK

DSTRIDE = 1 << 20
TLIN = EMBED_DIM * DSTRIDE

DT_HALF2 = 499968
DT_REM2 = NUM_TEAMS - 2 * DT_HALF2

_sc_mesh = plsc.VectorSubcoreMesh(
    core_axis_name="c", subcore_axis_name="s", num_cores=NC, num_subcores=NS
)


def _gather_body(tt_hbm, idxo_hbm, out_hbm, scr_hbm,
                 rem, idx_v, rows_v, sg0, sg1):
  c = lax.axis_index("c")
  s = lax.axis_index("s")

  d = c * D_SC + s // 2
  h = s % 2
  col0 = h * DT_HALF2
  pltpu.sync_copy(
      tt_hbm.at[d, pl.ds(col0, DT_HALF2)],
      scr_hbm.at[pl.ds(d * DSTRIDE + col0, DT_HALF2)],
  )

  @pl.when(h == 1)
  def _():
    pltpu.sync_copy(tt_hbm.at[d, pl.ds(2 * DT_HALF2, DT_REM2)], rem)
    pltpu.sync_copy(rem, scr_hbm.at[pl.ds(d * DSTRIDE + 2 * DT_HALF2, DT_REM2)])

  plsc.subcore_barrier()

  pltpu.sync_copy(idxo_hbm.at[s, pl.ds(c * D_SC, D_SC)], idx_v)
  sgs = (sg0, sg1)

  def wave(j):
    return [
        pltpu.make_async_copy(
            scr_hbm.at[idx_v.at[dl, j]],
            rows_v.at[dl, pl.ds(j * CHUNK, CHUNK)],
            sgs[j % 2],
        )
        for dl in range(D_SC)
    ]

  prev = wave(0)
  for cp in prev:
    cp.start()
  for j in range(1, KJ):
    cur = wave(j)
    for cp in cur:
      cp.start()
    for cp in prev:
      cp.wait()
    prev = cur
  for cp in prev:
    cp.wait()
  pltpu.sync_copy(
      rows_v, out_hbm.at[pl.ds(c * D_SC, D_SC), pl.ds(s * T_W, T_W)]
  )


_gather = pl.kernel(
    _gather_body,
    out_type=[
        jax.ShapeDtypeStruct((EMBED_DIM, B2), jnp.float32),
        jax.ShapeDtypeStruct((TLIN,), jnp.float32),
    ],
    mesh=_sc_mesh,
    scratch_types=[
        pltpu.VMEM((DT_REM2,), jnp.float32),
        pltpu.VMEM((D_SC, KJ, CHUNK), jnp.int32),
        pltpu.VMEM((D_SC, T_W), jnp.float32),
        pltpu.SemaphoreType.DMA,
        pltpu.SemaphoreType.DMA,
    ],
    compiler_params=pltpu.CompilerParams(use_tc_tiling_on_sc=True),
)

BM = 16384
NB = BATCH // BM


def _mlp_body(gft, home, away, w1t, b1c, w2t, b2c, w3ht, w3at, w3ft, b3c,
              w4t, b4c, out):
  h = jnp.maximum(
      jnp.dot(w1t[:], gft[:], preferred_element_type=jnp.float32) + b1c[:], 0.0
  )
  fo = jnp.dot(w2t[:], h, preferred_element_type=jnp.float32) + b2c[:]
  pre = (
      jnp.dot(w3ht[:], home[:], preferred_element_type=jnp.float32)
      + jnp.dot(w3at[:], away[:], preferred_element_type=jnp.float32)
      + jnp.dot(w3ft[:], fo, preferred_element_type=jnp.float32)
      + b3c[:]
  )
  g = jnp.maximum(pre, 0.0)
  out[:] = jnp.dot(w4t[:], g, preferred_element_type=jnp.float32) + b4c[:]


def _full(shape):
  return pl.BlockSpec(shape, lambda i: (0,) * len(shape))


_mlp = pl.pallas_call(
    _mlp_body,
    grid=(NB,),
    in_specs=[
        pl.BlockSpec((NUM_FEATURES, BM), lambda i: (0, i)),
        pl.BlockSpec((EMBED_DIM, BM), lambda i: (0, i)),
        pl.BlockSpec((EMBED_DIM, BM), lambda i: (0, i + NB)),
        _full((16, NUM_FEATURES)),
        _full((16, 1)),
        _full((8, 16)),
        _full((8, 1)),
        _full((8, EMBED_DIM)),
        _full((8, EMBED_DIM)),
        _full((8, 8)),
        _full((8, 1)),
        _full((1, 8)),
        _full((1, 1)),
    ],
    out_specs=pl.BlockSpec((1, BM), lambda i: (0, i)),
    out_shape=jax.ShapeDtypeStruct((1, BATCH), jnp.float32),
)


@jax.jit
def kernel(home_team_id, away_team_id, game_features, table,
           W1, b1, W2, b2, W3, b3, W4, b4):
  idx = jnp.concatenate(
      [home_team_id.astype(jnp.int32), away_team_id.astype(jnp.int32)]
  ).reshape(NS, 1, KJ, CHUNK)
  dim_offs = (jnp.arange(EMBED_DIM, dtype=jnp.int32) * DSTRIDE)[
      None, :, None, None
  ]
  idxo = idx + dim_offs
  embt, _ = _gather(table.T, idxo)
  out_t = _mlp(
      game_features.T,
      embt,
      embt,
      W1.T,
      b1.reshape(16, 1),
      W2.T,
      b2.reshape(8, 1),
      W3[:EMBED_DIM].T,
      W3[EMBED_DIM : 2 * EMBED_DIM].T,
      W3[2 * EMBED_DIM :].T,
      b3.reshape(8, 1),
      W4.T,
      b4.reshape(1, 1),
  )
  return out_t.reshape(BATCH, 1)


# speedup vs baseline: 21.3426x; 21.3426x over previous
import functools

import jax
import jax.numpy as jnp
from jax import lax
from jax.experimental import pallas as pl
from jax.experimental.pallas import tpu as pltpu
from jax.experimental.pallas import tpu_sc as plsc

NUM_TEAMS = 1000000
EMBED_DIM = 16
NUM_FEATURES = 22
BATCH = 16384

NC, NS = 2, 16
NW = NC * NS
CHUNK = 128
B2 = 2 * BATCH
D_SC = EMBED_DIM // NC
T_W = B2 // NS
KJ = T_W // CHUNK

DSTRIDE = 1 << 20
TLIN = EMBED_DIM * DSTRIDE

DT_C = 41600
DT_K = 12
DT_HALF = DT_C * DT_K
DT_REM = NUM_TEAMS - 2 * DT_HALF

_sc_mesh = plsc.VectorSubcoreMesh(
    core_axis_name="c", subcore_axis_name="s", num_cores=NC, num_subcores=NS
)


def _gather_body(tt_hbm, idxo_hbm, out_hbm, scr_hbm,
                 buf0, buf1, rem, idx_v, rows_v,
                 sr0, sr1, sw0, sw1, sg0, sg1):
  c = lax.axis_index("c")
  s = lax.axis_index("s")

  d = c * D_SC + s // 2
  h = s % 2
  col0 = h * DT_HALF
  dst0 = d * DSTRIDE + col0
  bufs = (buf0, buf1)
  srs = (sr0, sr1)
  sws = (sw0, sw1)

  def rd(k):
    return pltpu.make_async_copy(
        tt_hbm.at[d, pl.ds(col0 + k * DT_C, DT_C)], bufs[k % 2], srs[k % 2]
    )

  def wr(k):
    return pltpu.make_async_copy(
        bufs[k % 2], scr_hbm.at[pl.ds(dst0 + k * DT_C, DT_C)], sws[k % 2]
    )

  rd(0).start()
  rd(1).start()
  for k in range(DT_K):
    rd(k).wait()
    wr(k).start()
    if k + 2 < DT_K:
      wr(k).wait()
      rd(k + 2).start()
  wr(DT_K - 2).wait()
  wr(DT_K - 1).wait()

  @pl.when(h == 1)
  def _():
    pltpu.sync_copy(tt_hbm.at[d, pl.ds(2 * DT_HALF, DT_REM)], rem)
    pltpu.sync_copy(rem, scr_hbm.at[pl.ds(d * DSTRIDE + 2 * DT_HALF, DT_REM)])

  plsc.subcore_barrier()

  pltpu.sync_copy(idxo_hbm.at[s, pl.ds(c * D_SC, D_SC)], idx_v)
  sgs = (sg0, sg1)

  def wave(j):
    return [
        pltpu.make_async_copy(
            scr_hbm.at[idx_v.at[dl, j]],
            rows_v.at[dl, pl.ds(j * CHUNK, CHUNK)],
            sgs[j % 2],
        )
        for dl in range(D_SC)
    ]

  prev = wave(0)
  for cp in prev:
    cp.start()
  for j in range(1, KJ):
    cur = wave(j)
    for cp in cur:
      cp.start()
    for cp in prev:
      cp.wait()
    prev = cur
  for cp in prev:
    cp.wait()
  pltpu.sync_copy(
      rows_v, out_hbm.at[pl.ds(c * D_SC, D_SC), pl.ds(s * T_W, T_W)]
  )


_gather = pl.kernel(
    _gather_body,
    out_type=[
        jax.ShapeDtypeStruct((EMBED_DIM, B2), jnp.float32),
        jax.ShapeDtypeStruct((TLIN,), jnp.float32),
    ],
    mesh=_sc_mesh,
    scratch_types=[
        pltpu.VMEM((DT_C,), jnp.float32),
        pltpu.VMEM((DT_C,), jnp.float32),
        pltpu.VMEM((DT_REM,), jnp.float32),
        pltpu.VMEM((D_SC, KJ, CHUNK), jnp.int32),
        pltpu.VMEM((D_SC, T_W), jnp.float32),
        pltpu.SemaphoreType.DMA,
        pltpu.SemaphoreType.DMA,
        pltpu.SemaphoreType.DMA,
        pltpu.SemaphoreType.DMA,
        pltpu.SemaphoreType.DMA,
        pltpu.SemaphoreType.DMA,
    ],
    compiler_params=pltpu.CompilerParams(use_tc_tiling_on_sc=True),
)

BM = 16384
NB = BATCH // BM


def _mlp_body(gft, home, away, w1t, b1c, w2t, b2c, w3ht, w3at, w3ft, b3c,
              w4t, b4c, out):
  h = jnp.maximum(
      jnp.dot(w1t[:], gft[:], preferred_element_type=jnp.float32) + b1c[:], 0.0
  )
  fo = jnp.dot(w2t[:], h, preferred_element_type=jnp.float32) + b2c[:]
  pre = (
      jnp.dot(w3ht[:], home[:], preferred_element_type=jnp.float32)
      + jnp.dot(w3at[:], away[:], preferred_element_type=jnp.float32)
      + jnp.dot(w3ft[:], fo, preferred_element_type=jnp.float32)
      + b3c[:]
  )
  g = jnp.maximum(pre, 0.0)
  out[:] = jnp.dot(w4t[:], g, preferred_element_type=jnp.float32) + b4c[:]


def _full(shape):
  return pl.BlockSpec(shape, lambda i: (0,) * len(shape))


_mlp = pl.pallas_call(
    _mlp_body,
    grid=(NB,),
    in_specs=[
        pl.BlockSpec((NUM_FEATURES, BM), lambda i: (0, i)),
        pl.BlockSpec((EMBED_DIM, BM), lambda i: (0, i)),
        pl.BlockSpec((EMBED_DIM, BM), lambda i: (0, i + NB)),
        _full((16, NUM_FEATURES)),
        _full((16, 1)),
        _full((8, 16)),
        _full((8, 1)),
        _full((8, EMBED_DIM)),
        _full((8, EMBED_DIM)),
        _full((8, 8)),
        _full((8, 1)),
        _full((1, 8)),
        _full((1, 1)),
    ],
    out_specs=pl.BlockSpec((1, BM), lambda i: (0, i)),
    out_shape=jax.ShapeDtypeStruct((1, BATCH), jnp.float32),
)


@jax.jit
def kernel(home_team_id, away_team_id, game_features, table,
           W1, b1, W2, b2, W3, b3, W4, b4):
  idx = jnp.concatenate(
      [home_team_id.astype(jnp.int32), away_team_id.astype(jnp.int32)]
  ).reshape(NS, 1, KJ, CHUNK)
  dim_offs = (jnp.arange(EMBED_DIM, dtype=jnp.int32) * DSTRIDE)[
      None, :, None, None
  ]
  idxo = idx + dim_offs
  embt, _ = _gather(table.T, idxo)
  out_t = _mlp(
      game_features.T,
      embt,
      embt,
      W1.T,
      b1.reshape(16, 1),
      W2.T,
      b2.reshape(8, 1),
      W3[:EMBED_DIM].T,
      W3[EMBED_DIM : 2 * EMBED_DIM].T,
      W3[2 * EMBED_DIM :].T,
      b3.reshape(8, 1),
      W4.T,
      b4.reshape(1, 1),
  )
  return out_t.reshape(BATCH, 1)
